# initial kernel scaffold (unmeasured)
import jax
import jax.numpy as jnp
from jax import lax
from jax.experimental import pallas as pl
from jax.experimental.pallas import tpu as pltpu

N_DEV = 4


def kernel(x, w_mat):
    m, k = x.shape
    _, n = w_mat.shape
    chunk = m // N_DEV

    def body(x_ref, w_ref, out_ref, acc_ref, comm_ref, send_sems, recv_sems):
        my = lax.axis_index("i")
        left = lax.rem(my + N_DEV - 1, N_DEV)
        right = lax.rem(my + 1, N_DEV)

        barrier_sem = pltpu.get_barrier_semaphore()
        for nbr in (left, right):
            pl.semaphore_signal(
                barrier_sem, inc=1,
                device_id=(nbr,), device_id_type=pl.DeviceIdType.MESH,
            )
        pl.semaphore_wait(barrier_sem, 2)

        for c in range(N_DEV):
            acc_ref[c] = jnp.dot(
                x_ref[pl.ds(c * chunk, chunk), :], w_ref[...],
                preferred_element_type=jnp.float32,
            )

        for s in range(N_DEV - 1):
            send_idx = lax.rem(my - s + N_DEV, N_DEV)
            recv_idx = lax.rem(my - s - 1 + N_DEV, N_DEV)
            rdma = pltpu.make_async_remote_copy(
                src_ref=acc_ref.at[send_idx],
                dst_ref=comm_ref.at[s],
                send_sem=send_sems.at[s],
                recv_sem=recv_sems.at[s],
                device_id=(right,),
                device_id_type=pl.DeviceIdType.MESH,
            )
            rdma.start()
            rdma.wait()
            acc_ref[recv_idx] = acc_ref[recv_idx] + comm_ref[s]

        for g in range(N_DEV - 1):
            send_idx = lax.rem(my + 1 - g + N_DEV, N_DEV)
            rdma = pltpu.make_async_remote_copy(
                src_ref=acc_ref.at[send_idx],
                dst_ref=acc_ref.at[send_idx],
                send_sem=send_sems.at[N_DEV - 1 + g],
                recv_sem=recv_sems.at[N_DEV - 1 + g],
                device_id=(right,),
                device_id_type=pl.DeviceIdType.MESH,
            )
            rdma.start()
            rdma.wait()

        for c in range(N_DEV):
            out_ref[pl.ds(c * chunk, chunk), :] = acc_ref[c]

    return pl.pallas_call(
        body,
        out_shape=jax.ShapeDtypeStruct((m, n), jnp.float32),
        in_specs=[
            pl.BlockSpec(memory_space=pltpu.VMEM),
            pl.BlockSpec(memory_space=pltpu.VMEM),
        ],
        out_specs=pl.BlockSpec(memory_space=pltpu.VMEM),
        scratch_shapes=[
            pltpu.VMEM((N_DEV, chunk, n), jnp.float32),
            pltpu.VMEM((N_DEV - 1, chunk, n), jnp.float32),
            pltpu.SemaphoreType.DMA((2 * (N_DEV - 1),)),
            pltpu.SemaphoreType.DMA((2 * (N_DEV - 1),)),
        ],
        compiler_params=pltpu.CompilerParams(collective_id=0),
    )(x, w_mat)


# baseline (device time: 311797 ns/iter reference)
import jax
import jax.numpy as jnp
from jax import lax
from jax.experimental import pallas as pl
from jax.experimental.pallas import tpu as pltpu

N_DEV = 4


def kernel(x, w_mat):
    m, k = x.shape
    _, n = w_mat.shape
    chunk = m // N_DEV

    def body(x_ref, w_ref, out_ref, acc_ref, comm_ref, send_sems, recv_sems):
        my = lax.axis_index("i")
        left = lax.rem(my + N_DEV - 1, N_DEV)
        right = lax.rem(my + 1, N_DEV)

        barrier_sem = pltpu.get_barrier_semaphore()
        for nbr in (left, right):
            pl.semaphore_signal(
                barrier_sem, inc=1,
                device_id=(nbr,), device_id_type=pl.DeviceIdType.MESH,
            )
        pl.semaphore_wait(barrier_sem, 2)

        for c in range(N_DEV):
            acc_ref[c] = jnp.dot(
                x_ref[pl.ds(c * chunk, chunk), :], w_ref[...],
                preferred_element_type=jnp.float32,
            )

        for s in range(N_DEV - 1):
            send_idx = lax.rem(my - s + N_DEV, N_DEV)
            recv_idx = lax.rem(my - s - 1 + N_DEV, N_DEV)
            rdma = pltpu.make_async_remote_copy(
                src_ref=acc_ref.at[send_idx],
                dst_ref=comm_ref.at[s],
                send_sem=send_sems.at[s],
                recv_sem=recv_sems.at[s],
                device_id=(right,),
                device_id_type=pl.DeviceIdType.MESH,
            )
            rdma.start()
            rdma.wait()
            acc_ref[recv_idx] = acc_ref[recv_idx] + comm_ref[s]

        for g in range(N_DEV - 1):
            send_idx = lax.rem(my + 1 - g + N_DEV, N_DEV)
            rdma = pltpu.make_async_remote_copy(
                src_ref=acc_ref.at[send_idx],
                dst_ref=acc_ref.at[send_idx],
                send_sem=send_sems.at[N_DEV - 1 + g],
                recv_sem=recv_sems.at[N_DEV - 1 + g],
                device_id=(right,),
                device_id_type=pl.DeviceIdType.MESH,
            )
            rdma.start()
            rdma.wait()

        for c in range(N_DEV):
            out_ref[pl.ds(c * chunk, chunk), :] = acc_ref[c]

    return pl.pallas_call(
        body,
        out_shape=jax.ShapeDtypeStruct((m, n), jnp.float32),
        in_specs=[
            pl.BlockSpec(memory_space=pltpu.VMEM),
            pl.BlockSpec(memory_space=pltpu.VMEM),
        ],
        out_specs=pl.BlockSpec(memory_space=pltpu.VMEM),
        scratch_shapes=[
            pltpu.VMEM((N_DEV, chunk, n), jnp.float32),
            pltpu.VMEM((N_DEV - 1, chunk, n), jnp.float32),
            pltpu.SemaphoreType.DMA((2 * (N_DEV - 1),)),
            pltpu.SemaphoreType.DMA((2 * (N_DEV - 1),)),
        ],
        compiler_params=pltpu.CompilerParams(
            collective_id=0, vmem_limit_bytes=100 * 1024 * 1024,
        ),
    )(x, w_mat)


# device time: 177099 ns/iter; 1.7606x vs baseline; 1.7606x over previous
import jax
import jax.numpy as jnp
from jax import lax
from jax.experimental import pallas as pl
from jax.experimental.pallas import tpu as pltpu

N_DEV = 4
N_CH = 2 * N_DEV


def kernel(x, w_mat):
    m, k = x.shape
    _, n = w_mat.shape
    ch = m // N_CH

    def body(x_ref, w_ref, out_ref, acc_ref, comm_ref, send_sems, recv_sems):
        my = lax.axis_index("i")
        left = lax.rem(my + N_DEV - 1, N_DEV)
        right = lax.rem(my + 1, N_DEV)

        barrier_sem = pltpu.get_barrier_semaphore()
        for nbr in (left, right):
            pl.semaphore_signal(
                barrier_sem, inc=1,
                device_id=(nbr,), device_id_type=pl.DeviceIdType.MESH,
            )
        pl.semaphore_wait(barrier_sem, 2)

        for c in range(N_CH):
            acc_ref[c] = jnp.dot(
                x_ref[pl.ds(c * ch, ch), :], w_ref[...],
                preferred_element_type=jnp.float32,
            )

        for s in range(N_DEV - 1):
            send_a = 2 * lax.rem(my - s + N_DEV, N_DEV)
            recv_a = 2 * lax.rem(my - s - 1 + N_DEV, N_DEV)
            send_b = 2 * lax.rem(my + s, N_DEV) + 1
            recv_b = 2 * lax.rem(my + s + 1, N_DEV) + 1
            rdma_a = pltpu.make_async_remote_copy(
                src_ref=acc_ref.at[send_a],
                dst_ref=comm_ref.at[0, s],
                send_sem=send_sems.at[0, s],
                recv_sem=recv_sems.at[0, s],
                device_id=(right,),
                device_id_type=pl.DeviceIdType.MESH,
            )
            rdma_b = pltpu.make_async_remote_copy(
                src_ref=acc_ref.at[send_b],
                dst_ref=comm_ref.at[1, s],
                send_sem=send_sems.at[1, s],
                recv_sem=recv_sems.at[1, s],
                device_id=(left,),
                device_id_type=pl.DeviceIdType.MESH,
            )
            rdma_a.start()
            rdma_b.start()
            rdma_a.wait()
            rdma_b.wait()
            acc_ref[recv_a] = acc_ref[recv_a] + comm_ref[0, s]
            acc_ref[recv_b] = acc_ref[recv_b] + comm_ref[1, s]

        for g in range(N_DEV - 1):
            send_a = 2 * lax.rem(my + 1 - g + N_DEV, N_DEV)
            send_b = 2 * lax.rem(my - 1 + g + N_DEV, N_DEV) + 1
            rdma_a = pltpu.make_async_remote_copy(
                src_ref=acc_ref.at[send_a],
                dst_ref=acc_ref.at[send_a],
                send_sem=send_sems.at[0, N_DEV - 1 + g],
                recv_sem=recv_sems.at[0, N_DEV - 1 + g],
                device_id=(right,),
                device_id_type=pl.DeviceIdType.MESH,
            )
            rdma_b = pltpu.make_async_remote_copy(
                src_ref=acc_ref.at[send_b],
                dst_ref=acc_ref.at[send_b],
                send_sem=send_sems.at[1, N_DEV - 1 + g],
                recv_sem=recv_sems.at[1, N_DEV - 1 + g],
                device_id=(left,),
                device_id_type=pl.DeviceIdType.MESH,
            )
            rdma_a.start()
            rdma_b.start()
            rdma_a.wait()
            rdma_b.wait()

        for c in range(N_CH):
            out_ref[pl.ds(c * ch, ch), :] = acc_ref[c]

    return pl.pallas_call(
        body,
        out_shape=jax.ShapeDtypeStruct((m, n), jnp.float32),
        in_specs=[
            pl.BlockSpec(memory_space=pltpu.VMEM),
            pl.BlockSpec(memory_space=pltpu.VMEM),
        ],
        out_specs=pl.BlockSpec(memory_space=pltpu.VMEM),
        scratch_shapes=[
            pltpu.VMEM((N_CH, ch, n), jnp.float32),
            pltpu.VMEM((2, N_DEV - 1, ch, n), jnp.float32),
            pltpu.SemaphoreType.DMA((2, 2 * (N_DEV - 1))),
            pltpu.SemaphoreType.DMA((2, 2 * (N_DEV - 1))),
        ],
        compiler_params=pltpu.CompilerParams(
            collective_id=0, vmem_limit_bytes=100 * 1024 * 1024,
        ),
    )(x, w_mat)


# device time: 163521 ns/iter; 1.9068x vs baseline; 1.0830x over previous
import jax
import jax.numpy as jnp
from jax import lax
from jax.experimental import pallas as pl
from jax.experimental.pallas import tpu as pltpu

N_DEV = 4
SUB = 128
HOPS = N_DEV - 1


def kernel(x, w_mat):
    m, k = x.shape
    _, n = w_mat.shape

    def body(x_ref, w_ref, out_ref, comm_ref, send_sems, recv_sems):
        my = lax.axis_index("i")
        left = lax.rem(my + N_DEV - 1, N_DEV)
        right = lax.rem(my + 1, N_DEV)
        nbr = (right, left)

        barrier_sem = pltpu.get_barrier_semaphore()
        for b in (left, right):
            pl.semaphore_signal(
                barrier_sem, inc=1,
                device_id=(b,), device_id_type=pl.DeviceIdType.MESH,
            )
        pl.semaphore_wait(barrier_sem, 2)

        def rows(slot):
            return pl.ds(slot * SUB, SUB)

        def gemm(slot):
            out_ref[rows(slot), :] = jnp.dot(
                x_ref[rows(slot), :], w_ref[...],
                preferred_element_type=jnp.float32,
            )

        def rs_send_slot(d, s, u):
            j = lax.rem(my - s + N_DEV, N_DEV) if d == 0 else lax.rem(my + s, N_DEV)
            return 4 * j + 2 * d + u

        def rs_recv_slot(d, s, u):
            j = (
                lax.rem(my - s - 1 + N_DEV, N_DEV)
                if d == 0
                else lax.rem(my + s + 1, N_DEV)
            )
            return 4 * j + 2 * d + u

        def ag_send_slot(d, g, u):
            j = (
                lax.rem(my + 1 - g + N_DEV, N_DEV)
                if d == 0
                else lax.rem(my - 1 + g + N_DEV, N_DEV)
            )
            return 4 * j + 2 * d + u

        all_rdmas = []

        def rs_rdma(d, s, u):
            i = 2 * s + u
            r = pltpu.make_async_remote_copy(
                src_ref=out_ref.at[rows(rs_send_slot(d, s, u)), :],
                dst_ref=comm_ref.at[d, i],
                send_sem=send_sems.at[d, i],
                recv_sem=recv_sems.at[d, i],
                device_id=(nbr[d],),
                device_id_type=pl.DeviceIdType.MESH,
            )
            all_rdmas.append(r)
            return r

        def ag_rdma(d, g, u):
            i = 2 * HOPS + 2 * g + u
            slot = ag_send_slot(d, g, u)
            r = pltpu.make_async_remote_copy(
                src_ref=out_ref.at[rows(slot), :],
                dst_ref=out_ref.at[rows(slot), :],
                send_sem=send_sems.at[d, i],
                recv_sem=recv_sems.at[d, i],
                device_id=(nbr[d],),
                device_id_type=pl.DeviceIdType.MESH,
            )
            all_rdmas.append(r)
            return r

        for t in range(4):
            gemm(4 * my + t)

        live = {}
        for u in (0, 1):
            for d in (0, 1):
                live[(d, 0, u)] = rs_rdma(d, 0, u)
                live[(d, 0, u)].start()

        for q in (1, 2, 3):
            for t in range(4):
                gemm(4 * lax.rem(my + q, N_DEV) + t)

        for s in range(HOPS):
            for u in (0, 1):
                for d in (0, 1):
                    live[(d, s, u)].wait_recv()
                    dst = rows(rs_recv_slot(d, s, u))
                    out_ref[dst, :] = out_ref[dst, :] + comm_ref[d, 2 * s + u]
                    if s + 1 < HOPS:
                        live[(d, s + 1, u)] = rs_rdma(d, s + 1, u)
                        live[(d, s + 1, u)].start()
                    else:
                        live[(d, HOPS + 0, u)] = ag_rdma(d, 0, u)
                        live[(d, HOPS + 0, u)].start()

        for g in range(HOPS):
            for u in (0, 1):
                for d in (0, 1):
                    live[(d, HOPS + g, u)].wait_recv()
                    if g + 1 < HOPS:
                        live[(d, HOPS + g + 1, u)] = ag_rdma(d, g + 1, u)
                        live[(d, HOPS + g + 1, u)].start()

        for r in all_rdmas:
            r.wait_send()

    return pl.pallas_call(
        body,
        out_shape=jax.ShapeDtypeStruct((m, n), jnp.float32),
        in_specs=[
            pl.BlockSpec(memory_space=pltpu.VMEM),
            pl.BlockSpec(memory_space=pltpu.VMEM),
        ],
        out_specs=pl.BlockSpec(memory_space=pltpu.VMEM),
        scratch_shapes=[
            pltpu.VMEM((2, 2 * HOPS, SUB, n), jnp.float32),
            pltpu.SemaphoreType.DMA((2, 4 * HOPS)),
            pltpu.SemaphoreType.DMA((2, 4 * HOPS)),
        ],
        compiler_params=pltpu.CompilerParams(
            collective_id=0, vmem_limit_bytes=100 * 1024 * 1024,
        ),
    )(x, w_mat)
